# baseline (speedup 1.0000x reference)
import jax
import jax.numpy as jnp
from jax import lax
from jax.experimental import pallas as pl
from jax.experimental.pallas import tpu as pltpu

N = 8192
M = 8192
BJ = 1024


def _chamfer_kernel(x_ref, y_ref, out_ref, xm_ref, xn_ref, rowmin_ref,
                    colacc_ref):
    j = pl.program_id(0)
    nj = pl.num_programs(0)

    @pl.when(j == 0)
    def _init():
        x = x_ref[...]
        xm_ref[...] = x * -2.0
        xn_ref[...] = jnp.sum(x * x, axis=1, keepdims=True)
        rowmin_ref[...] = jnp.full_like(rowmin_ref, jnp.inf)
        colacc_ref[0, 0] = 0.0

    yb = y_ref[...]
    yn = lax.transpose(jnp.sum(yb * yb, axis=1, keepdims=True), (1, 0))
    cross = lax.dot_general(xm_ref[...], yb, (((1,), (1,)), ((), ())))
    d = (xn_ref[...] + yn) + cross

    rowmin_ref[...] = jnp.minimum(rowmin_ref[...], jnp.min(d, axis=1, keepdims=True))
    colacc_ref[0, 0] += jnp.sum(jnp.min(d, axis=0))

    @pl.when(j == nj - 1)
    def _finish():
        mean_row = jnp.sum(rowmin_ref[...]) / N
        mean_col = colacc_ref[0, 0] / M
        out_ref[...] = jnp.full((1, 1), (mean_row + mean_col) * 0.5, jnp.float32)


def kernel(pred_positions, target_positions):
    out = pl.pallas_call(
        _chamfer_kernel,
        grid=(M // BJ,),
        in_specs=[
            pl.BlockSpec((N, 3), lambda j: (0, 0)),
            pl.BlockSpec((BJ, 3), lambda j: (j, 0)),
        ],
        out_specs=pl.BlockSpec((1, 1), lambda j: (0, 0)),
        out_shape=jax.ShapeDtypeStruct((1, 1), jnp.float32),
        scratch_shapes=[
            pltpu.VMEM((N, 3), jnp.float32),
            pltpu.VMEM((N, 1), jnp.float32),
            pltpu.VMEM((N, 1), jnp.float32),
            pltpu.SMEM((1, 1), jnp.float32),
        ],
    )(pred_positions, target_positions)
    return out[0, 0]
